# Initial kernel scaffold; baseline (speedup 1.0000x reference)
import jax
import jax.numpy as jnp
from jax.experimental import pallas as pl


def kernel(features, edge_index, W1, b1, W_l0, b_l0, W_l1, b_l1, W2, b2):
    raise NotImplementedError("write your pallas kernel here")


# speedup vs baseline: 19.0756x; 19.0756x over previous
import functools

import jax
import jax.numpy as jnp
from jax import lax
from jax.experimental import pallas as pl
from jax.experimental.pallas import tpu as pltpu
from jax.experimental.pallas import tpu_sc as plsc

N = 10000
E = 320000
D = 128
NCLS = 40

NC = 2
NS = 16
NW = NC * NS
CHUNK = 128
NCHUNK = 79
E_PAD = NW * NCHUNK * CHUNK
N_PAD = 10240
TRASH = N_PAD - N
ROWS_PER_TILE = N_PAD // NS
ZCH = 128

_mesh = plsc.VectorSubcoreMesh(core_axis_name="c", subcore_axis_name="s")


def _fill_rows(ref, nrows, ncols16, val):
    def body(r, _):
        for k in range(ncols16):
            ref[r, pl.ds(k * 16, 16)] = jnp.full((16,), val, jnp.float32)
        return 0
    lax.fori_loop(0, nrows, body, 0)


@functools.partial(
    pl.kernel,
    mesh=_mesh,
    out_type=jax.ShapeDtypeStruct((NC, N_PAD), jnp.float32),
    scratch_types=[
        pltpu.VMEM((NCHUNK, CHUNK), jnp.int32),
        pltpu.VMEM((CHUNK,), jnp.float32),
        pltpu.VMEM_SHARED((N_PAD,), jnp.float32),
    ],
)
def _deg_kernel(dst_hbm, out_hbm, dst_v, ones_v, deg_sh):
    c = lax.axis_index("c")
    s = lax.axis_index("s")
    w = s * NC + c
    base = s * ROWS_PER_TILE

    for k in range(CHUNK // 16):
        ones_v[pl.ds(k * 16, 16)] = jnp.zeros((16,), jnp.float32)
    for k in range(ROWS_PER_TILE // ZCH):
        pltpu.sync_copy(ones_v, deg_sh.at[pl.ds(base + k * ZCH, ZCH)])
    for k in range(CHUNK // 16):
        ones_v[pl.ds(k * 16, 16)] = jnp.ones((16,), jnp.float32)
    plsc.subcore_barrier()

    pltpu.sync_copy(dst_hbm.at[w], dst_v)

    def body(j, _):
        pltpu.sync_copy(ones_v, deg_sh.at[dst_v.at[j]], add=True)
        return 0
    lax.fori_loop(0, NCHUNK, body, 0)

    plsc.subcore_barrier()
    pltpu.sync_copy(deg_sh.at[pl.ds(base, ROWS_PER_TILE)],
                    out_hbm.at[c].at[pl.ds(base, ROWS_PER_TILE)])


@functools.partial(
    pl.kernel,
    mesh=_mesh,
    out_type=jax.ShapeDtypeStruct((NC, N_PAD, D), jnp.float32),
    scratch_types=[
        pltpu.VMEM((NCHUNK, CHUNK), jnp.int32),
        pltpu.VMEM((NCHUNK, CHUNK), jnp.int32),
        pltpu.VMEM((CHUNK, D), jnp.float32),
        pltpu.VMEM_SHARED((N_PAD, D), jnp.float32),
        pltpu.SemaphoreType.DMA,
    ],
)
def _agg_kernel(hs_hbm, src_hbm, dst_hbm, out_hbm,
                src_v, dst_v, rows_v, acc_sh, sem):
    c = lax.axis_index("c")
    s = lax.axis_index("s")
    w = s * NC + c
    base = s * ROWS_PER_TILE

    _fill_rows(rows_v, CHUNK, D // 16, 0.0)
    for k in range(ROWS_PER_TILE // ZCH):
        pltpu.sync_copy(rows_v, acc_sh.at[pl.ds(base + k * ZCH, ZCH)])
    plsc.subcore_barrier()

    pltpu.sync_copy(src_hbm.at[w], src_v)
    pltpu.sync_copy(dst_hbm.at[w], dst_v)

    def body(j, _):
        pltpu.async_copy(hs_hbm.at[src_v.at[j]], rows_v, sem).wait()
        pltpu.sync_copy(rows_v, acc_sh.at[dst_v.at[j]], add=True)
        return 0
    lax.fori_loop(0, NCHUNK, body, 0)

    plsc.subcore_barrier()
    pltpu.sync_copy(acc_sh.at[pl.ds(base, ROWS_PER_TILE)],
                    out_hbm.at[c].at[pl.ds(base, ROWS_PER_TILE)])


RB = 1000


def _inv_from_deg(deg_ref):
    return lax.rsqrt(jnp.maximum(deg_ref[...], 1.0))


def _proj_body(x_ref, w_ref, b_ref, deg_ref, o_ref):
    h = jnp.dot(x_ref[...], w_ref[...],
                preferred_element_type=jnp.float32) + b_ref[...]
    o_ref[...] = h * _inv_from_deg(deg_ref)


def _mid_body(agg_ref, deg_ref, w_ref, b_ref, o_ref):
    inv = _inv_from_deg(deg_ref)
    agg = (agg_ref[0] + agg_ref[1]) * inv
    h = jnp.maximum(
        jnp.dot(agg, w_ref[...], preferred_element_type=jnp.float32)
        + b_ref[...], 0.0)
    o_ref[...] = h * inv


def _final_body(agg_ref, deg_ref, w_ref, b_ref, w2_ref, b2_ref, o_ref):
    inv = _inv_from_deg(deg_ref)
    agg = (agg_ref[0] + agg_ref[1]) * inv
    h = jnp.maximum(
        jnp.dot(agg, w_ref[...], preferred_element_type=jnp.float32)
        + b_ref[...], 0.0)
    o_ref[...] = jnp.dot(h, w2_ref[...],
                         preferred_element_type=jnp.float32) + b2_ref[...]


_deg_spec = pl.BlockSpec((RB, 1), lambda i: (i, 0))
_row_spec = pl.BlockSpec((RB, D), lambda i: (i, 0))
_agg_spec = pl.BlockSpec((2, RB, D), lambda i: (0, i, 0))
_w_spec = pl.BlockSpec((D, D), lambda i: (0, 0))
_b_spec = pl.BlockSpec((1, D), lambda i: (0, 0))

_proj = pl.pallas_call(
    _proj_body, grid=(N // RB,),
    in_specs=[_row_spec, _w_spec, _b_spec, _deg_spec],
    out_specs=_row_spec,
    out_shape=jax.ShapeDtypeStruct((N, D), jnp.float32),
)

_mid = pl.pallas_call(
    _mid_body, grid=(N // RB,),
    in_specs=[_agg_spec, _deg_spec, _w_spec, _b_spec],
    out_specs=_row_spec,
    out_shape=jax.ShapeDtypeStruct((N, D), jnp.float32),
)

_final = pl.pallas_call(
    _final_body, grid=(N // RB,),
    in_specs=[_agg_spec, _deg_spec, _w_spec, _b_spec,
              pl.BlockSpec((D, NCLS), lambda i: (0, 0)),
              pl.BlockSpec((1, NCLS), lambda i: (0, 0))],
    out_specs=pl.BlockSpec((RB, NCLS), lambda i: (i, 0)),
    out_shape=jax.ShapeDtypeStruct((N, NCLS), jnp.float32),
)


def kernel(features, edge_index, W1, b1, W_l0, b_l0, W_l1, b_l1, W2, b2):
    src = edge_index[0]
    dst = edge_index[1]
    pad_i = jnp.arange(E_PAD - E, dtype=jnp.int32)
    src3 = jnp.concatenate([src, pad_i % TRASH]).reshape(NW, NCHUNK, CHUNK)
    dst3 = jnp.concatenate([dst, N + (pad_i % TRASH)]).reshape(
        NW, NCHUNK, CHUNK)

    degp = _deg_kernel(dst3)
    deg = (degp[0] + degp[1])[:N].reshape(N, 1)
    hs = _proj(features, W1, b1.reshape(1, D), deg)
    aggp = _agg_kernel(hs, src3, dst3)
    hs = _mid(aggp, deg, W_l0, b_l0.reshape(1, D))
    aggp = _agg_kernel(hs, src3, dst3)
    return _final(aggp, deg, W_l1, b_l1.reshape(1, D),
                  W2, b2.reshape(1, NCLS))


# speedup vs baseline: 27.0011x; 1.4155x over previous
import functools

import jax
import jax.numpy as jnp
from jax import lax
from jax.experimental import pallas as pl
from jax.experimental.pallas import tpu as pltpu
from jax.experimental.pallas import tpu_sc as plsc

N = 10000
E = 320000
D = 128
NCLS = 40

NC = 2
NS = 16
NW = NC * NS
CHUNK = 128
NCHUNK = 80
HALF = NCHUNK // 2
E_PAD = NW * NCHUNK * CHUNK
N_PAD = 10240
TRASH = N_PAD - N
ROWS_PER_TILE = N_PAD // NS
ZCH = CHUNK

_mesh = plsc.VectorSubcoreMesh(core_axis_name="c", subcore_axis_name="s")


def _fill_rows(ref, nrows, ncols16, val):
    def body(r, _):
        for k in range(ncols16):
            ref[r, pl.ds(k * 16, 16)] = jnp.full((16,), val, jnp.float32)
        return 0
    lax.fori_loop(0, nrows, body, 0)


@functools.partial(
    pl.kernel,
    mesh=_mesh,
    out_type=jax.ShapeDtypeStruct((NC, N_PAD), jnp.float32),
    scratch_types=[
        pltpu.VMEM((NCHUNK, CHUNK), jnp.int32),
        pltpu.VMEM((CHUNK,), jnp.float32),
        pltpu.VMEM_SHARED((N_PAD,), jnp.float32),
    ],
)
def _deg_kernel(dst_hbm, out_hbm, dst_v, ones_v, deg_sh):
    c = lax.axis_index("c")
    s = lax.axis_index("s")
    w = s * NC + c
    base = s * ROWS_PER_TILE

    for k in range(CHUNK // 16):
        ones_v[pl.ds(k * 16, 16)] = jnp.zeros((16,), jnp.float32)
    for k in range(ROWS_PER_TILE // ZCH):
        pltpu.sync_copy(ones_v, deg_sh.at[pl.ds(base + k * ZCH, ZCH)])
    for k in range(CHUNK // 16):
        ones_v[pl.ds(k * 16, 16)] = jnp.ones((16,), jnp.float32)
    plsc.subcore_barrier()

    pltpu.sync_copy(dst_hbm.at[w], dst_v)

    def body(j, _):
        pltpu.sync_copy(ones_v, deg_sh.at[dst_v.at[j]], add=True)
        return 0
    lax.fori_loop(0, NCHUNK, body, 0)

    plsc.subcore_barrier()
    pltpu.sync_copy(deg_sh.at[pl.ds(base, ROWS_PER_TILE)],
                    out_hbm.at[c].at[pl.ds(base, ROWS_PER_TILE)])


@functools.partial(
    pl.kernel,
    mesh=_mesh,
    out_type=jax.ShapeDtypeStruct((NC, N_PAD, D), jnp.float32),
    scratch_types=[
        pltpu.VMEM((HALF, CHUNK), jnp.int32),
        pltpu.VMEM((HALF, CHUNK), jnp.int32),
        pltpu.VMEM((CHUNK, D), jnp.float32),
        pltpu.VMEM((CHUNK, D), jnp.float32),
        pltpu.VMEM_SHARED((N_PAD, D), jnp.float32),
        pltpu.SemaphoreType.DMA,
        pltpu.SemaphoreType.DMA,
    ],
)
def _agg_kernel(hs_hbm, src_hbm, dst_hbm, out_hbm,
                src_v, dst_v, rows_a, rows_b, acc_sh, sem_a, sem_b):
    c = lax.axis_index("c")
    s = lax.axis_index("s")
    w = s * NC + c
    base = s * ROWS_PER_TILE

    _fill_rows(rows_a, CHUNK, D // 16, 0.0)
    for k in range(ROWS_PER_TILE // ZCH):
        pltpu.sync_copy(rows_a, acc_sh.at[pl.ds(base + k * ZCH, ZCH)])
    plsc.subcore_barrier()

    for h in range(2):
        pltpu.sync_copy(src_hbm.at[w].at[pl.ds(h * HALF, HALF)], src_v)
        pltpu.sync_copy(dst_hbm.at[w].at[pl.ds(h * HALF, HALF)], dst_v)
        pltpu.async_copy(hs_hbm.at[src_v.at[0]], rows_a, sem_a)

        def pair(i, _):
            ja = 2 * i
            jb = 2 * i + 1
            cp_b = pltpu.async_copy(hs_hbm.at[src_v.at[jb]], rows_b, sem_b)
            pltpu.make_async_copy(hs_hbm.at[src_v.at[ja]], rows_a,
                                  sem_a).wait()
            pltpu.sync_copy(rows_a, acc_sh.at[dst_v.at[ja]], add=True)

            @pl.when(jb + 1 < HALF)
            def _():
                pltpu.async_copy(hs_hbm.at[src_v.at[jb + 1]], rows_a, sem_a)
            cp_b.wait()
            pltpu.sync_copy(rows_b, acc_sh.at[dst_v.at[jb]], add=True)
            return 0
        lax.fori_loop(0, HALF // 2, pair, 0)

    plsc.subcore_barrier()
    pltpu.sync_copy(acc_sh.at[pl.ds(base, ROWS_PER_TILE)],
                    out_hbm.at[c].at[pl.ds(base, ROWS_PER_TILE)])


RB = 1000


def _inv_from_deg(deg_ref):
    return lax.rsqrt(jnp.maximum(deg_ref[...], 1.0))


def _proj_body(x_ref, w_ref, b_ref, deg_ref, o_ref):
    h = jnp.dot(x_ref[...], w_ref[...],
                preferred_element_type=jnp.float32) + b_ref[...]
    o_ref[...] = h * _inv_from_deg(deg_ref)


def _mid_body(agg_ref, deg_ref, w_ref, b_ref, o_ref):
    inv = _inv_from_deg(deg_ref)
    agg = (agg_ref[0] + agg_ref[1]) * inv
    h = jnp.maximum(
        jnp.dot(agg, w_ref[...], preferred_element_type=jnp.float32)
        + b_ref[...], 0.0)
    o_ref[...] = h * inv


def _final_body(agg_ref, deg_ref, w_ref, b_ref, w2_ref, b2_ref, o_ref):
    inv = _inv_from_deg(deg_ref)
    agg = (agg_ref[0] + agg_ref[1]) * inv
    h = jnp.maximum(
        jnp.dot(agg, w_ref[...], preferred_element_type=jnp.float32)
        + b_ref[...], 0.0)
    o_ref[...] = jnp.dot(h, w2_ref[...],
                         preferred_element_type=jnp.float32) + b2_ref[...]


_deg_spec = pl.BlockSpec((RB, 1), lambda i: (i, 0))
_row_spec = pl.BlockSpec((RB, D), lambda i: (i, 0))
_agg_spec = pl.BlockSpec((2, RB, D), lambda i: (0, i, 0))
_w_spec = pl.BlockSpec((D, D), lambda i: (0, 0))
_b_spec = pl.BlockSpec((1, D), lambda i: (0, 0))

_proj = pl.pallas_call(
    _proj_body, grid=(N // RB,),
    in_specs=[_row_spec, _w_spec, _b_spec, _deg_spec],
    out_specs=_row_spec,
    out_shape=jax.ShapeDtypeStruct((N, D), jnp.float32),
)

_mid = pl.pallas_call(
    _mid_body, grid=(N // RB,),
    in_specs=[_agg_spec, _deg_spec, _w_spec, _b_spec],
    out_specs=_row_spec,
    out_shape=jax.ShapeDtypeStruct((N, D), jnp.float32),
)

_final = pl.pallas_call(
    _final_body, grid=(N // RB,),
    in_specs=[_agg_spec, _deg_spec, _w_spec, _b_spec,
              pl.BlockSpec((D, NCLS), lambda i: (0, 0)),
              pl.BlockSpec((1, NCLS), lambda i: (0, 0))],
    out_specs=pl.BlockSpec((RB, NCLS), lambda i: (i, 0)),
    out_shape=jax.ShapeDtypeStruct((N, NCLS), jnp.float32),
)


def kernel(features, edge_index, W1, b1, W_l0, b_l0, W_l1, b_l1, W2, b2):
    src = edge_index[0]
    dst = edge_index[1]
    pad_i = jnp.arange(E_PAD - E, dtype=jnp.int32)
    src3 = jnp.concatenate([src, pad_i % TRASH]).reshape(NW, NCHUNK, CHUNK)
    dst3 = jnp.concatenate([dst, N + (pad_i % TRASH)]).reshape(
        NW, NCHUNK, CHUNK)

    degp = _deg_kernel(dst3)
    deg = (degp[0] + degp[1])[:N].reshape(N, 1)
    hs = _proj(features, W1, b1.reshape(1, D), deg)
    aggp = _agg_kernel(hs, src3, dst3)
    hs = _mid(aggp, deg, W_l0, b_l0.reshape(1, D))
    aggp = _agg_kernel(hs, src3, dst3)
    return _final(aggp, deg, W_l1, b_l1.reshape(1, D),
                  W2, b2.reshape(1, NCLS))


# speedup vs baseline: 28.2577x; 1.0465x over previous
import functools

import jax
import jax.numpy as jnp
from jax import lax
from jax.experimental import pallas as pl
from jax.experimental.pallas import tpu as pltpu
from jax.experimental.pallas import tpu_sc as plsc

N = 10000
E = 320000
D = 128
NCLS = 40

NC = 2
NS = 16
NW = NC * NS
CHUNK = 128
NCHUNK = 80
ACH = 64
ANCH = 160
QW = 40
E_PAD = NW * NCHUNK * CHUNK
N_PAD = 10240
TRASH = N_PAD - N
ROWS_PER_TILE = N_PAD // NS
ZCH = CHUNK

_mesh = plsc.VectorSubcoreMesh(core_axis_name="c", subcore_axis_name="s")


def _fill_rows(ref, nrows, ncols16, val):
    def body(r, _):
        for k in range(ncols16):
            ref[r, pl.ds(k * 16, 16)] = jnp.full((16,), val, jnp.float32)
        return 0
    lax.fori_loop(0, nrows, body, 0)


@functools.partial(
    pl.kernel,
    mesh=_mesh,
    out_type=jax.ShapeDtypeStruct((NC, N_PAD), jnp.float32),
    scratch_types=[
        pltpu.VMEM((NCHUNK, CHUNK), jnp.int32),
        pltpu.VMEM((CHUNK,), jnp.float32),
        pltpu.VMEM_SHARED((N_PAD,), jnp.float32),
    ],
)
def _deg_kernel(dst_hbm, out_hbm, dst_v, ones_v, deg_sh):
    c = lax.axis_index("c")
    s = lax.axis_index("s")
    w = s * NC + c
    base = s * ROWS_PER_TILE

    for k in range(CHUNK // 16):
        ones_v[pl.ds(k * 16, 16)] = jnp.zeros((16,), jnp.float32)
    for k in range(ROWS_PER_TILE // ZCH):
        pltpu.sync_copy(ones_v, deg_sh.at[pl.ds(base + k * ZCH, ZCH)])
    for k in range(CHUNK // 16):
        ones_v[pl.ds(k * 16, 16)] = jnp.ones((16,), jnp.float32)
    plsc.subcore_barrier()

    pltpu.sync_copy(dst_hbm.at[w], dst_v)

    def body(j, _):
        pltpu.sync_copy(ones_v, deg_sh.at[dst_v.at[j]], add=True)
        return 0
    lax.fori_loop(0, NCHUNK, body, 0)

    plsc.subcore_barrier()
    pltpu.sync_copy(deg_sh.at[pl.ds(base, ROWS_PER_TILE)],
                    out_hbm.at[c].at[pl.ds(base, ROWS_PER_TILE)])


@functools.partial(
    pl.kernel,
    mesh=_mesh,
    out_type=jax.ShapeDtypeStruct((NC, N_PAD, D), jnp.float32),
    scratch_types=[
        pltpu.VMEM((QW, ACH), jnp.int32),
        pltpu.VMEM((QW, ACH), jnp.int32),
        pltpu.VMEM((ACH, D), jnp.float32),
        pltpu.VMEM((ACH, D), jnp.float32),
        pltpu.VMEM((ACH, D), jnp.float32),
        pltpu.VMEM((ACH, D), jnp.float32),
        pltpu.VMEM_SHARED((N_PAD, D), jnp.float32),
        pltpu.SemaphoreType.DMA,
        pltpu.SemaphoreType.DMA,
        pltpu.SemaphoreType.DMA,
        pltpu.SemaphoreType.DMA,
    ],
)
def _agg_kernel(hs_hbm, src_hbm, dst_hbm, out_hbm,
                src_v, dst_v, r0, r1, r2, r3, acc_sh, s0, s1, s2, s3):
    c = lax.axis_index("c")
    s = lax.axis_index("s")
    w = s * NC + c
    base = s * ROWS_PER_TILE
    rows = (r0, r1, r2, r3)
    sems = (s0, s1, s2, s3)

    _fill_rows(r0, ACH, D // 16, 0.0)
    for k in range(ROWS_PER_TILE // ACH):
        pltpu.sync_copy(r0, acc_sh.at[pl.ds(base + k * ACH, ACH)])
    plsc.subcore_barrier()

    for h in range(4):
        pltpu.sync_copy(src_hbm.at[w].at[pl.ds(h * QW, QW)], src_v)
        pltpu.sync_copy(dst_hbm.at[w].at[pl.ds(h * QW, QW)], dst_v)
        for b in range(3):
            pltpu.async_copy(hs_hbm.at[src_v.at[b]], rows[b], sems[b])

        def quad(i, _):
            for b in range(4):
                j = 4 * i + b
                nxt = j + 3

                @pl.when(nxt < QW)
                def _():
                    pltpu.async_copy(hs_hbm.at[src_v.at[nxt]],
                                     rows[(b + 3) % 4], sems[(b + 3) % 4])
                pltpu.make_async_copy(hs_hbm.at[src_v.at[j]],
                                      rows[b], sems[b]).wait()
                pltpu.sync_copy(rows[b], acc_sh.at[dst_v.at[j]], add=True)
            return 0
        lax.fori_loop(0, QW // 4, quad, 0)

    plsc.subcore_barrier()
    pltpu.sync_copy(acc_sh.at[pl.ds(base, ROWS_PER_TILE)],
                    out_hbm.at[c].at[pl.ds(base, ROWS_PER_TILE)])


RB = 1000


def _inv_from_deg(deg_ref):
    return lax.rsqrt(jnp.maximum(deg_ref[...], 1.0))


def _proj_body(x_ref, w_ref, b_ref, deg_ref, o_ref):
    h = jnp.dot(x_ref[...], w_ref[...],
                preferred_element_type=jnp.float32) + b_ref[...]
    o_ref[...] = h * _inv_from_deg(deg_ref)


def _mid_body(agg_ref, deg_ref, w_ref, b_ref, o_ref):
    inv = _inv_from_deg(deg_ref)
    agg = (agg_ref[0] + agg_ref[1]) * inv
    h = jnp.maximum(
        jnp.dot(agg, w_ref[...], preferred_element_type=jnp.float32)
        + b_ref[...], 0.0)
    o_ref[...] = h * inv


def _final_body(agg_ref, deg_ref, w_ref, b_ref, w2_ref, b2_ref, o_ref):
    inv = _inv_from_deg(deg_ref)
    agg = (agg_ref[0] + agg_ref[1]) * inv
    h = jnp.maximum(
        jnp.dot(agg, w_ref[...], preferred_element_type=jnp.float32)
        + b_ref[...], 0.0)
    o_ref[...] = jnp.dot(h, w2_ref[...],
                         preferred_element_type=jnp.float32) + b2_ref[...]


_deg_spec = pl.BlockSpec((RB, 1), lambda i: (i, 0))
_row_spec = pl.BlockSpec((RB, D), lambda i: (i, 0))
_agg_spec = pl.BlockSpec((2, RB, D), lambda i: (0, i, 0))
_w_spec = pl.BlockSpec((D, D), lambda i: (0, 0))
_b_spec = pl.BlockSpec((1, D), lambda i: (0, 0))

_proj = pl.pallas_call(
    _proj_body, grid=(N // RB,),
    in_specs=[_row_spec, _w_spec, _b_spec, _deg_spec],
    out_specs=_row_spec,
    out_shape=jax.ShapeDtypeStruct((N, D), jnp.float32),
)

_mid = pl.pallas_call(
    _mid_body, grid=(N // RB,),
    in_specs=[_agg_spec, _deg_spec, _w_spec, _b_spec],
    out_specs=_row_spec,
    out_shape=jax.ShapeDtypeStruct((N, D), jnp.float32),
)

_final = pl.pallas_call(
    _final_body, grid=(N // RB,),
    in_specs=[_agg_spec, _deg_spec, _w_spec, _b_spec,
              pl.BlockSpec((D, NCLS), lambda i: (0, 0)),
              pl.BlockSpec((1, NCLS), lambda i: (0, 0))],
    out_specs=pl.BlockSpec((RB, NCLS), lambda i: (i, 0)),
    out_shape=jax.ShapeDtypeStruct((N, NCLS), jnp.float32),
)


def kernel(features, edge_index, W1, b1, W_l0, b_l0, W_l1, b_l1, W2, b2):
    src = edge_index[0]
    dst = edge_index[1]
    pad_i = jnp.arange(E_PAD - E, dtype=jnp.int32)
    src3 = jnp.concatenate([src, pad_i % TRASH]).reshape(NW, ANCH, ACH)
    dst3 = jnp.concatenate([dst, N + (pad_i % TRASH)]).reshape(
        NW, ANCH, ACH)

    degp = _deg_kernel(dst3.reshape(NW, NCHUNK, CHUNK))
    deg = (degp[0] + degp[1])[:N].reshape(N, 1)
    hs = _proj(features, W1, b1.reshape(1, D), deg)
    aggp = _agg_kernel(hs, src3, dst3)
    hs = _mid(aggp, deg, W_l0, b_l0.reshape(1, D))
    aggp = _agg_kernel(hs, src3, dst3)
    return _final(aggp, deg, W_l1, b_l1.reshape(1, D),
                  W2, b2.reshape(1, NCLS))
